# Initial kernel scaffold; baseline (speedup 1.0000x reference)
import jax
import jax.numpy as jnp
from jax.experimental import pallas as pl


def kernel(idx, weight):
    raise NotImplementedError("write your pallas kernel here")


# speedup vs baseline: 1.5357x; 1.5357x over previous
import functools

import jax
import jax.numpy as jnp
from jax import lax
from jax.experimental import pallas as pl
from jax.experimental.pallas import tpu as pltpu
from jax.experimental.pallas import tpu_sc as plsc

_VOCAB = 100000
_HIDDEN = 1024
_B = 4 * 2048
_NC = 2
_NS = 16
_NW = _NC * _NS
_BPW = _B // _NW
_K = 32
_NCHUNK = _BPW // _K

_mesh = plsc.VectorSubcoreMesh(core_axis_name="c", subcore_axis_name="s")


@functools.partial(
    pl.kernel,
    mesh=_mesh,
    out_type=jax.ShapeDtypeStruct((_B, _HIDDEN), jnp.float32),
    scratch_types=[
        pltpu.VMEM((_NCHUNK, _K), jnp.int32),
        pltpu.VMEM((_K, _HIDDEN), jnp.float32),
        pltpu.VMEM((_K, _HIDDEN), jnp.float32),
        pltpu.SemaphoreType.DMA,
        pltpu.SemaphoreType.DMA,
    ],
)
def _gather_kernel(idx_hbm, table_hbm, out_hbm, idx_v, buf0, buf1, sem0, sem1):
    wid = lax.axis_index("s") * _NC + lax.axis_index("c")
    base = wid * _BPW
    pltpu.sync_copy(idx_hbm.at[wid], idx_v)
    pltpu.async_copy(table_hbm.at[idx_v.at[0]], buf0, sem0)
    for c in range(_NCHUNK):
        buf, sem = (buf0, sem0) if c % 2 == 0 else (buf1, sem1)
        nbuf, nsem = (buf1, sem1) if c % 2 == 0 else (buf0, sem0)
        if c + 1 < _NCHUNK:
            pltpu.async_copy(table_hbm.at[idx_v.at[c + 1]], nbuf, nsem)
        pltpu.make_async_copy(table_hbm.at[idx_v.at[c]], buf, sem).wait()
        pltpu.sync_copy(buf, out_hbm.at[pl.ds(base + c * _K, _K)])


def kernel(idx, weight):
    batch, seq = idx.shape
    idx_grid = idx.reshape(_NW, _NCHUNK, _K)
    out = _gather_kernel(idx_grid, weight)
    return out.reshape(batch, seq, weight.shape[1])


# speedup vs baseline: 1.5403x; 1.0030x over previous
import functools

import jax
import jax.numpy as jnp
from jax import lax
from jax.experimental import pallas as pl
from jax.experimental.pallas import tpu as pltpu
from jax.experimental.pallas import tpu_sc as plsc

_VOCAB = 100000
_HIDDEN = 1024
_B = 4 * 2048
_NC = 2
_NS = 16
_NW = _NC * _NS
_BPW = _B // _NW
_K = 32
_NCHUNK = _BPW // _K

_mesh = plsc.VectorSubcoreMesh(core_axis_name="c", subcore_axis_name="s")


@functools.partial(
    pl.kernel,
    mesh=_mesh,
    out_type=jax.ShapeDtypeStruct((_B, _HIDDEN), jnp.float32),
    scratch_types=[
        pltpu.VMEM((_NCHUNK, _K), jnp.int32),
        pltpu.VMEM((_K, _HIDDEN), jnp.float32),
        pltpu.VMEM((_K, _HIDDEN), jnp.float32),
        pltpu.VMEM((_K, _HIDDEN), jnp.float32),
        pltpu.SemaphoreType.DMA,
        pltpu.SemaphoreType.DMA,
        pltpu.SemaphoreType.DMA,
        pltpu.SemaphoreType.DMA,
        pltpu.SemaphoreType.DMA,
        pltpu.SemaphoreType.DMA,
    ],
)
def _gather_kernel(idx_hbm, table_hbm, out_hbm, idx_v,
                   b0, b1, b2, g0, g1, g2, o0, o1, o2):
    wid = lax.axis_index("s") * _NC + lax.axis_index("c")
    base = wid * _BPW
    bufs = (b0, b1, b2)
    gsems = (g0, g1, g2)
    osems = (o0, o1, o2)
    pltpu.sync_copy(idx_hbm.at[wid], idx_v)
    pltpu.async_copy(table_hbm.at[idx_v.at[0]], bufs[0], gsems[0])
    pltpu.async_copy(table_hbm.at[idx_v.at[1]], bufs[1], gsems[1])
    for c in range(_NCHUNK):
        r = c % 3
        out_slice = out_hbm.at[pl.ds(base + c * _K, _K)]
        pltpu.make_async_copy(table_hbm.at[idx_v.at[c]], bufs[r], gsems[r]).wait()
        pltpu.async_copy(bufs[r], out_slice, osems[r])
        if c + 2 < _NCHUNK:
            nr = (c + 2) % 3
            if c - 1 >= 0:
                prev_slice = out_hbm.at[pl.ds(base + (c - 1) * _K, _K)]
                pltpu.make_async_copy(bufs[nr], prev_slice, osems[nr]).wait()
            pltpu.async_copy(table_hbm.at[idx_v.at[c + 2]], bufs[nr], gsems[nr])
    for c in (_NCHUNK - 2, _NCHUNK - 1):
        r = c % 3
        out_slice = out_hbm.at[pl.ds(base + c * _K, _K)]
        pltpu.make_async_copy(bufs[r], out_slice, osems[r]).wait()


def kernel(idx, weight):
    batch, seq = idx.shape
    idx_grid = idx.reshape(_NW, _NCHUNK, _K)
    out = _gather_kernel(idx_grid, weight)
    return out.reshape(batch, seq, weight.shape[1])
